# baseline (speedup 1.0000x reference)
import functools

import jax
import jax.numpy as jnp
from jax import lax
from jax.experimental import pallas as pl
from jax.experimental.pallas import tpu as pltpu

_K_NN = 6
_EPS = 1e-6
_HIP_L, _HIP_R = 11, 12


def _tag_body(rm_ref, bmgl_ref, ieff_ref, x_ref, o_ref, *, C, T, V, M):
    S = C * T
    P = V * M
    xb = x_ref[0]

    cols0 = _HIP_L * M
    pos_rows = []
    for c in range(3):
        hc = xb[c * T:(c + 1) * T, cols0:cols0 + 2 * M]
        root = 0.5 * (hc[:, :M] + hc[:, M:2 * M])
        pos_rows.append(jnp.mean(root, axis=0))
    pos = jnp.stack(pos_rows, axis=0)

    diff = pos[:, :, None] - pos[:, None, :]
    d = jnp.sqrt(jnp.sum(diff * diff, axis=0))

    rowi = lax.broadcasted_iota(jnp.int32, (M, M), 0)
    coli = lax.broadcasted_iota(jnp.int32, (M, M), 1)
    rankT = jnp.zeros((M, M), jnp.int32)
    for u in range(M):
        du = d[u:u + 1, :]
        lt = (du < d).astype(jnp.int32)
        eq = ((du == d) & (u < rowi)).astype(jnp.int32)
        rankT = rankT + lt + eq
    eye = (rowi == coli).astype(jnp.float32)
    AT = (rankT < _K_NN).astype(jnp.float32) + eye
    ATb = AT.astype(jnp.bfloat16)

    rm = rm_ref[...]
    wmid = jnp.dot(rm, ATb, preferred_element_type=jnp.float32)
    w3 = lax.dot_general(wmid.astype(jnp.bfloat16), rm, (((1,), (1,)), ((), ())),
                         preferred_element_type=jnp.float32)
    wf = (w3 * bmgl_ref[...] + ieff_ref[...]).astype(jnp.bfloat16)

    o_ref[0] = jnp.dot(xb.astype(jnp.bfloat16), wf,
                       preferred_element_type=jnp.float32)


def kernel(x, lambda_fuse, tag_gate):
    N, C, T, V, M = x.shape
    S, P = C * T, V * M
    x2 = x.reshape(N, S, P)
    gl = (jax.nn.sigmoid(tag_gate) * lambda_fuse).astype(jnp.float32)

    pcol = jnp.arange(P, dtype=jnp.int32)
    rm = (pcol[:, None] % M == jnp.arange(M, dtype=jnp.int32)[None, :])
    rm = rm.astype(jnp.bfloat16)
    blk = (pcol[:, None] // M == pcol[None, :] // M).astype(jnp.float32)
    dinv = 1.0 / jnp.sqrt(jnp.float32(_K_NN + 1) + _EPS)
    bmgl = blk * (gl * dinv * dinv)
    ieff = (1.0 - gl) * jnp.eye(P, dtype=jnp.float32)

    body = functools.partial(_tag_body, C=C, T=T, V=V, M=M)
    out = pl.pallas_call(
        body,
        grid=(N,),
        in_specs=[
            pl.BlockSpec((P, M), lambda n: (0, 0)),
            pl.BlockSpec((P, P), lambda n: (0, 0)),
            pl.BlockSpec((P, P), lambda n: (0, 0)),
            pl.BlockSpec((1, S, P), lambda n: (n, 0, 0)),
        ],
        out_specs=pl.BlockSpec((1, S, P), lambda n: (n, 0, 0)),
        out_shape=jax.ShapeDtypeStruct((N, S, P), jnp.float32),
        compiler_params=pltpu.CompilerParams(
            dimension_semantics=("arbitrary",),
        ),
    )(rm, bmgl, ieff, x2)
    return out.reshape(N, C, T, V, M)
